# baseline (speedup 1.0000x reference)
import functools

import jax
import jax.numpy as jnp
from jax import lax
from jax.experimental import pallas as pl
from jax.experimental.pallas import tpu as pltpu
from jax.experimental.pallas import tpu_sc as plsc

N = 10000
E = 160000
T = 8
D = 256
HALF = 128
NBLK = 10
RB = N // NBLK

NSC = 2
NTILE = 16
CHUNK = 128
NCH = 80
EPT = NCH * CHUNK
ACC_ROWS = 10240
RPT = ACC_ROWS // NTILE


def _gru_body(*refs):
    outs = refs[20:]
    for c in range(2):
        (w0, wz, uz, bz, wr, ur, br, wh, uh, bh) = refs[10 * c:10 * (c + 1)]
        H = w0[...]
        Wz, Uz, Wr, Ur = wz[...], uz[...], wr[...], ur[...]
        Wh = wh[...]
        Uh = uh[...]
        bzv = bz[...]
        brv = br[...]
        bhv = bh[...]

        def mm(a, b):
            return jnp.dot(a, b, preferred_element_type=jnp.float32)

        for _ in range(T):
            z = jax.nn.sigmoid(mm(Wz, H) + mm(Uz, H) + bzv)
            r = jax.nn.sigmoid(mm(Wr, H) + mm(Ur, H) + brv)
            ht = jnp.tanh(mm(Wh, H) + mm(Uh, r * H) + bhv)
            H = (1.0 - z) * H + z * ht
        outs[c][...] = H


def _gru_evolve(params0, params1):
    out = pl.pallas_call(
        _gru_body,
        out_shape=[jax.ShapeDtypeStruct((D, D), jnp.float32)] * 2,
    )(*params0, *params1)
    return out


def _stage_a_body(x_ref, h_ref, deg_ref, out_ref):
    xb = x_ref[...]
    d = deg_ref[0, :, :] + deg_ref[1, :, :] + 1.0
    dinv = lax.rsqrt(jnp.maximum(d, 1.0))
    out_ref[0] = jnp.dot(xb, h_ref[...], preferred_element_type=jnp.float32) * dinv


def _stage_a(x_last, H0, deg2):
    return pl.pallas_call(
        _stage_a_body,
        grid=(NBLK, 2),
        in_specs=[
            pl.BlockSpec((RB, D), lambda i, h: (i, 0)),
            pl.BlockSpec((D, HALF), lambda i, h: (0, h)),
            pl.BlockSpec((2, RB, 1), lambda i, h: (0, i, 0)),
        ],
        out_specs=pl.BlockSpec((1, RB, HALF), lambda i, h: (h, i, 0)),
        out_shape=jax.ShapeDtypeStruct((2, N, HALF), jnp.float32),
    )(x_last, H0, deg2)


def _stage_b_body(agg_ref, hw_ref, h_ref, deg_ref, out_ref):
    d = deg_ref[0, :, :] + deg_ref[1, :, :] + 1.0
    dinv = lax.rsqrt(jnp.maximum(d, 1.0))
    o0 = jnp.maximum(dinv * (agg_ref[0] + hw_ref[0]), 0.0)
    o1 = jnp.maximum(dinv * (agg_ref[1] + hw_ref[1]), 0.0)
    acc = (jnp.dot(o0, h_ref[:HALF, :], preferred_element_type=jnp.float32)
           + jnp.dot(o1, h_ref[HALF:, :], preferred_element_type=jnp.float32))
    out_ref[0] = acc * dinv


def _stage_b(agg0, hw0p, H1, deg2):
    return pl.pallas_call(
        _stage_b_body,
        grid=(NBLK, 2),
        in_specs=[
            pl.BlockSpec((2, RB, HALF), lambda i, h: (0, i, 0)),
            pl.BlockSpec((2, RB, HALF), lambda i, h: (0, i, 0)),
            pl.BlockSpec((D, HALF), lambda i, h: (0, h)),
            pl.BlockSpec((2, RB, 1), lambda i, h: (0, i, 0)),
        ],
        out_specs=pl.BlockSpec((1, RB, HALF), lambda i, h: (h, i, 0)),
        out_shape=jax.ShapeDtypeStruct((2, N, HALF), jnp.float32),
    )(agg0, hw0p, H1, deg2)


def _stage_c_body(agg_ref, hw_ref, deg_ref, out_ref):
    d = deg_ref[0, :, :] + deg_ref[1, :, :] + 1.0
    dinv = lax.rsqrt(jnp.maximum(d, 1.0))
    out_ref[...] = jnp.maximum(dinv * (agg_ref[0] + hw_ref[0]), 0.0)


def _stage_c(agg1, hw1p, deg2):
    return pl.pallas_call(
        _stage_c_body,
        grid=(NBLK, 2),
        in_specs=[
            pl.BlockSpec((1, RB, HALF), lambda i, h: (h, i, 0)),
            pl.BlockSpec((1, RB, HALF), lambda i, h: (h, i, 0)),
            pl.BlockSpec((2, RB, 1), lambda i, h: (0, i, 0)),
        ],
        out_specs=pl.BlockSpec((RB, HALF), lambda i, h: (i, h)),
        out_shape=jax.ShapeDtypeStruct((N, D), jnp.float32),
    )(agg1, hw1p, deg2)


def _zero_rows(buf, nrows):
    zero16 = jnp.zeros((16,), jnp.float32)
    ncol = buf.shape[1] // 16

    def zrow(r, carry):
        for k in range(ncol):
            buf[r, pl.ds(k * 16, 16)] = zero16
        return carry

    lax.fori_loop(0, nrows, zrow, 0, unroll=False)


def _spmm_sc_body(table, srcidx, dstidx, out,
                  srcall, d0, d1, rows0, rows1, acc,
                  semi0, semi1, semg0, semg1, sems0, sems1):
    c = lax.axis_index("c")
    s = lax.axis_index("s")
    w = c * NTILE + s
    pltpu.async_copy(dstidx.at[s].at[0], d0, semi0)
    pltpu.async_copy(dstidx.at[s].at[1], d1, semi1)
    pltpu.sync_copy(srcidx.at[w], srcall)
    _zero_rows(rows0, CHUNK)
    for q in range(5):
        pltpu.sync_copy(rows0, acc.at[pl.ds(s * RPT + q * CHUNK, CHUNK)])
    plsc.subcore_barrier()

    pltpu.async_copy(table.at[srcall.at[0]], rows0, semg0)
    pltpu.async_copy(table.at[srcall.at[1]], rows1, semg1)
    pltpu.make_async_copy(dstidx.at[s].at[0], d0, semi0).wait()

    def _wait_rows(rbuf, sem):
        pltpu.make_async_copy(table.at[srcall.at[0]], rbuf, sem).wait()

    def _wait_idx(dbuf, sem):
        pltpu.make_async_copy(dstidx.at[s].at[0], dbuf, sem).wait()

    def _wait_scat(rbuf, dbuf, sem):
        pltpu.make_async_copy(rbuf, acc.at[dbuf], sem).wait()

    def body(g, carry):
        j = 2 * g
        _wait_rows(rows0, semg0)
        pltpu.async_copy(rows0, acc.at[d0], sems0, add=True)
        _wait_idx(d1, semi1)
        _wait_rows(rows1, semg1)
        pltpu.async_copy(rows1, acc.at[d1], sems1, add=True)
        _wait_scat(rows0, d0, sems0)
        pltpu.async_copy(dstidx.at[s].at[j + 2], d0, semi0)
        pltpu.async_copy(table.at[srcall.at[j + 2]], rows0, semg0)
        _wait_scat(rows1, d1, sems1)
        pltpu.async_copy(dstidx.at[s].at[j + 3], d1, semi1)
        pltpu.async_copy(table.at[srcall.at[j + 3]], rows1, semg1)
        _wait_idx(d0, semi0)
        return carry

    lax.fori_loop(0, (NCH - 2) // 2, body, 0, unroll=False)
    _wait_rows(rows0, semg0)
    pltpu.async_copy(rows0, acc.at[d0], sems0, add=True)
    _wait_idx(d1, semi1)
    _wait_rows(rows1, semg1)
    pltpu.async_copy(rows1, acc.at[d1], sems1, add=True)
    _wait_scat(rows0, d0, sems0)
    _wait_scat(rows1, d1, sems1)

    plsc.subcore_barrier()
    pltpu.sync_copy(acc.at[pl.ds(s * RPT, RPT)],
                    out.at[c].at[pl.ds(s * RPT, RPT)])


_spmm_call = pl.kernel(
    _spmm_sc_body,
    out_type=jax.ShapeDtypeStruct((NSC, ACC_ROWS, HALF), jnp.float32),
    mesh=plsc.VectorSubcoreMesh(core_axis_name="c", subcore_axis_name="s"),
    scratch_types=[
        pltpu.VMEM((NCH, CHUNK), jnp.int32),
        pltpu.VMEM((CHUNK,), jnp.int32),
        pltpu.VMEM((CHUNK,), jnp.int32),
        pltpu.VMEM((CHUNK, HALF), jnp.float32),
        pltpu.VMEM((CHUNK, HALF), jnp.float32),
        pltpu.VMEM_SHARED((ACC_ROWS, HALF), jnp.float32),
        pltpu.SemaphoreType.DMA,
        pltpu.SemaphoreType.DMA,
        pltpu.SemaphoreType.DMA,
        pltpu.SemaphoreType.DMA,
        pltpu.SemaphoreType.DMA,
        pltpu.SemaphoreType.DMA,
    ],
)


def _spmm_sc(table2, srcidx, dstidx):
    return _spmm_call(table2.reshape(NSC * N, HALF), srcidx, dstidx)[:, :N]


NSEG = EPT // NTILE


def _deg_sc_body(dstidx, degout, dstall, d0, ones, zbuf, degacc):
    c = lax.axis_index("c")
    s = lax.axis_index("s")
    pltpu.sync_copy(dstidx.at[s], dstall)
    one16 = jnp.ones((16,), jnp.float32)
    zero16 = jnp.zeros((16,), jnp.float32)
    for k in range(CHUNK // 16):
        ones[pl.ds(16 * k, 16)] = one16

    def zrow(r, carry):
        zbuf[pl.ds(r * 16, 16)] = zero16
        return carry

    lax.fori_loop(0, NSEG // 16, zrow, 0, unroll=False)
    pltpu.sync_copy(zbuf, degacc.at[pl.ds(s * NSEG, NSEG)])
    plsc.subcore_barrier()

    def chunk_body(j, carry):
        for k in range(CHUNK // 16):
            d0[pl.ds(16 * k, 16)] = dstall[j, pl.ds(16 * k, 16)]
        pltpu.sync_copy(ones, degacc.at[d0], add=True)
        return carry

    lax.fori_loop(c * (NCH // 2), (c + 1) * (NCH // 2), chunk_body, 0,
                  unroll=False)
    plsc.subcore_barrier()
    pltpu.sync_copy(degacc.at[pl.ds(s * NSEG, NSEG)],
                    degout.at[c].at[pl.ds(s * NSEG, NSEG)])


_deg_call = pl.kernel(
    _deg_sc_body,
    out_type=jax.ShapeDtypeStruct((NSC, EPT), jnp.float32),
    mesh=plsc.VectorSubcoreMesh(core_axis_name="c", subcore_axis_name="s"),
    scratch_types=[
        pltpu.VMEM((NCH, CHUNK), jnp.int32),
        pltpu.VMEM((CHUNK,), jnp.int32),
        pltpu.VMEM((CHUNK,), jnp.float32),
        pltpu.VMEM((NSEG,), jnp.float32),
        pltpu.VMEM_SHARED((EPT,), jnp.float32),
    ],
)


def _edge_layout(edge_index):
    src = edge_index[0].reshape(NTILE, E // NTILE)
    dst = edge_index[1].reshape(NTILE, E // NTILE)
    pad = EPT - E // NTILE
    srcp = jnp.pad(src, ((0, 0), (0, pad)))
    dstp = jnp.pad(dst, ((0, 0), (0, pad)), constant_values=N)
    base = jnp.array([0, N], jnp.int32)[:, None, None]
    srcidx = (srcp[None] + base).reshape(NSC * NTILE, NCH, CHUNK)
    dstidx = dstp.reshape(NTILE, NCH, CHUNK)
    return srcidx, dstidx


def _degree_sc(dstidx):
    degout = _deg_call(dstidx)
    return degout[:, :N, None]


def kernel(x, edge_index, W0_0, Wz_0, Uz_0, bz_0, Wr_0, Ur_0, br_0, Wh_0, Uh_0, bh_0,
           W0_1, Wz_1, Uz_1, bz_1, Wr_1, Ur_1, br_1, Wh_1, Uh_1, bh_1):
    src = edge_index[0]
    dst = edge_index[1]
    H0, H1 = _gru_evolve(
        (W0_0, Wz_0, Uz_0, bz_0, Wr_0, Ur_0, br_0, Wh_0, Uh_0, bh_0),
        (W0_1, Wz_1, Uz_1, bz_1, Wr_1, Ur_1, br_1, Wh_1, Uh_1, bh_1))
    srcidx, dstidx = _edge_layout(edge_index)
    deg2 = _degree_sc(dstidx)
    hw0p = _stage_a(x[:, T - 1, :], H0, deg2)
    agg0 = _spmm_sc(hw0p, srcidx, dstidx)
    hw1p = _stage_b(agg0, hw0p, H1, deg2)
    agg1 = _spmm_sc(hw1p, srcidx, dstidx)
    return _stage_c(agg1, hw1p, deg2)


# speedup vs baseline: 1.1021x; 1.1021x over previous
import functools

import jax
import jax.numpy as jnp
from jax import lax
from jax.experimental import pallas as pl
from jax.experimental.pallas import tpu as pltpu
from jax.experimental.pallas import tpu_sc as plsc

N = 10000
E = 160000
T = 8
D = 256
HALF = 128
NBLK = 10
RB = N // NBLK

NSC = 2
NTILE = 16
CHUNK = 128
NCH = 80
EPT = NCH * CHUNK
ACC_ROWS = 10240
RPT = ACC_ROWS // NTILE


def _gru_body(*refs):
    outs = refs[20:]
    for c in range(2):
        (w0, wz, uz, bz, wr, ur, br, wh, uh, bh) = refs[10 * c:10 * (c + 1)]
        H = w0[...]
        Wz, Uz, Wr, Ur = wz[...], uz[...], wr[...], ur[...]
        Wh = wh[...]
        Uh = uh[...]
        bzv = bz[...]
        brv = br[...]
        bhv = bh[...]

        def mm(a, b):
            return jnp.dot(a, b, preferred_element_type=jnp.float32)

        for _ in range(T):
            z = jax.nn.sigmoid(mm(Wz, H) + mm(Uz, H) + bzv)
            r = jax.nn.sigmoid(mm(Wr, H) + mm(Ur, H) + brv)
            ht = jnp.tanh(mm(Wh, H) + mm(Uh, r * H) + bhv)
            H = (1.0 - z) * H + z * ht
        outs[c][...] = H


def _gru_evolve(params0, params1):
    out = pl.pallas_call(
        _gru_body,
        out_shape=[jax.ShapeDtypeStruct((D, D), jnp.float32)] * 2,
    )(*params0, *params1)
    return out


def _stage_a_body(x_ref, h_ref, deg_ref, out_ref):
    xb = x_ref[...]
    d = deg_ref[0, :, :] + deg_ref[1, :, :] + 1.0
    dinv = lax.rsqrt(jnp.maximum(d, 1.0))
    out_ref[0] = jnp.dot(xb, h_ref[...], preferred_element_type=jnp.float32) * dinv


def _stage_a(x2d, H0, deg2):
    return pl.pallas_call(
        _stage_a_body,
        grid=(NBLK, 2),
        in_specs=[
            pl.BlockSpec((RB, D), lambda i, h: (i, T - 1)),
            pl.BlockSpec((D, HALF), lambda i, h: (0, h)),
            pl.BlockSpec((2, RB, 1), lambda i, h: (0, i, 0)),
        ],
        out_specs=pl.BlockSpec((1, RB, HALF), lambda i, h: (h, i, 0)),
        out_shape=jax.ShapeDtypeStruct((2, N, HALF), jnp.float32),
    )(x2d, H0, deg2)


def _stage_b_body(agg_ref, hw_ref, h_ref, deg_ref, out_ref):
    d = deg_ref[0, :, :] + deg_ref[1, :, :] + 1.0
    dinv = lax.rsqrt(jnp.maximum(d, 1.0))
    o0 = jnp.maximum(dinv * (agg_ref[0] + hw_ref[0]), 0.0)
    o1 = jnp.maximum(dinv * (agg_ref[1] + hw_ref[1]), 0.0)
    acc = (jnp.dot(o0, h_ref[:HALF, :], preferred_element_type=jnp.float32)
           + jnp.dot(o1, h_ref[HALF:, :], preferred_element_type=jnp.float32))
    out_ref[0] = acc * dinv


def _stage_b(agg0, hw0p, H1, deg2):
    return pl.pallas_call(
        _stage_b_body,
        grid=(NBLK, 2),
        in_specs=[
            pl.BlockSpec((2, RB, HALF), lambda i, h: (0, i, 0)),
            pl.BlockSpec((2, RB, HALF), lambda i, h: (0, i, 0)),
            pl.BlockSpec((D, HALF), lambda i, h: (0, h)),
            pl.BlockSpec((2, RB, 1), lambda i, h: (0, i, 0)),
        ],
        out_specs=pl.BlockSpec((1, RB, HALF), lambda i, h: (h, i, 0)),
        out_shape=jax.ShapeDtypeStruct((2, N, HALF), jnp.float32),
    )(agg0, hw0p, H1, deg2)


def _stage_c_body(agg_ref, hw_ref, deg_ref, out_ref):
    d = deg_ref[0, :, :] + deg_ref[1, :, :] + 1.0
    dinv = lax.rsqrt(jnp.maximum(d, 1.0))
    out_ref[...] = jnp.maximum(dinv * (agg_ref[0] + hw_ref[0]), 0.0)


def _stage_c(agg1, hw1p, deg2):
    return pl.pallas_call(
        _stage_c_body,
        grid=(NBLK, 2),
        in_specs=[
            pl.BlockSpec((1, RB, HALF), lambda i, h: (h, i, 0)),
            pl.BlockSpec((1, RB, HALF), lambda i, h: (h, i, 0)),
            pl.BlockSpec((2, RB, 1), lambda i, h: (0, i, 0)),
        ],
        out_specs=pl.BlockSpec((RB, HALF), lambda i, h: (i, h)),
        out_shape=jax.ShapeDtypeStruct((N, D), jnp.float32),
    )(agg1, hw1p, deg2)


def _zero_rows(buf, nrows):
    zero16 = jnp.zeros((16,), jnp.float32)
    ncol = buf.shape[1] // 16

    def zrow(r, carry):
        for k in range(ncol):
            buf[r, pl.ds(k * 16, 16)] = zero16
        return carry

    lax.fori_loop(0, nrows, zrow, 0, unroll=False)


def _spmm_sc_body(table, srcidx, dstidx, out,
                  srcall, d0, d1, rows0, rows1, acc,
                  semi0, semi1, semg0, semg1, sems0, sems1):
    c = lax.axis_index("c")
    s = lax.axis_index("s")
    w = c * NTILE + s
    pltpu.async_copy(dstidx.at[s].at[0], d0, semi0)
    pltpu.async_copy(dstidx.at[s].at[1], d1, semi1)
    pltpu.sync_copy(srcidx.at[w], srcall)
    _zero_rows(rows0, CHUNK)
    for q in range(5):
        pltpu.sync_copy(rows0, acc.at[pl.ds(s * RPT + q * CHUNK, CHUNK)])
    plsc.subcore_barrier()

    pltpu.async_copy(table.at[srcall.at[0]], rows0, semg0)
    pltpu.async_copy(table.at[srcall.at[1]], rows1, semg1)
    pltpu.make_async_copy(dstidx.at[s].at[0], d0, semi0).wait()

    def _wait_rows(rbuf, sem):
        pltpu.make_async_copy(table.at[srcall.at[0]], rbuf, sem).wait()

    def _wait_idx(dbuf, sem):
        pltpu.make_async_copy(dstidx.at[s].at[0], dbuf, sem).wait()

    def body(g, carry):
        j = 2 * g
        _wait_rows(rows0, semg0)
        pltpu.sync_copy(rows0, acc.at[d0], add=True)
        pltpu.async_copy(dstidx.at[s].at[j + 2], d0, semi0)
        pltpu.async_copy(table.at[srcall.at[j + 2]], rows0, semg0)
        _wait_idx(d1, semi1)
        _wait_rows(rows1, semg1)
        pltpu.sync_copy(rows1, acc.at[d1], add=True)
        pltpu.async_copy(dstidx.at[s].at[j + 3], d1, semi1)
        pltpu.async_copy(table.at[srcall.at[j + 3]], rows1, semg1)
        _wait_idx(d0, semi0)
        return carry

    lax.fori_loop(0, (NCH - 2) // 2, body, 0, unroll=False)
    _wait_rows(rows0, semg0)
    pltpu.sync_copy(rows0, acc.at[d0], add=True)
    _wait_idx(d1, semi1)
    _wait_rows(rows1, semg1)
    pltpu.sync_copy(rows1, acc.at[d1], add=True)

    plsc.subcore_barrier()
    pltpu.sync_copy(acc.at[pl.ds(s * RPT, RPT)],
                    out.at[c].at[pl.ds(s * RPT, RPT)])


_spmm_call = pl.kernel(
    _spmm_sc_body,
    out_type=jax.ShapeDtypeStruct((NSC, ACC_ROWS, HALF), jnp.float32),
    mesh=plsc.VectorSubcoreMesh(core_axis_name="c", subcore_axis_name="s"),
    scratch_types=[
        pltpu.VMEM((NCH, CHUNK), jnp.int32),
        pltpu.VMEM((CHUNK,), jnp.int32),
        pltpu.VMEM((CHUNK,), jnp.int32),
        pltpu.VMEM((CHUNK, HALF), jnp.float32),
        pltpu.VMEM((CHUNK, HALF), jnp.float32),
        pltpu.VMEM_SHARED((ACC_ROWS, HALF), jnp.float32),
        pltpu.SemaphoreType.DMA,
        pltpu.SemaphoreType.DMA,
        pltpu.SemaphoreType.DMA,
        pltpu.SemaphoreType.DMA,
        pltpu.SemaphoreType.DMA,
        pltpu.SemaphoreType.DMA,
    ],
)


def _spmm_sc(table2, srcidx, dstidx):
    return _spmm_call(table2.reshape(NSC * N, HALF), srcidx, dstidx)


NSEG = EPT // NTILE


def _deg_sc_body(dstidx, degout, dstall, d0, ones, zbuf, degacc):
    c = lax.axis_index("c")
    s = lax.axis_index("s")
    pltpu.sync_copy(dstidx.at[s], dstall)
    one16 = jnp.ones((16,), jnp.float32)
    zero16 = jnp.zeros((16,), jnp.float32)
    for k in range(CHUNK // 16):
        ones[pl.ds(16 * k, 16)] = one16

    def zrow(r, carry):
        zbuf[pl.ds(r * 16, 16)] = zero16
        return carry

    lax.fori_loop(0, NSEG // 16, zrow, 0, unroll=False)
    pltpu.sync_copy(zbuf, degacc.at[pl.ds(s * NSEG, NSEG)])
    plsc.subcore_barrier()

    def chunk_body(j, carry):
        for k in range(CHUNK // 16):
            d0[pl.ds(16 * k, 16)] = dstall[j, pl.ds(16 * k, 16)]
        pltpu.sync_copy(ones, degacc.at[d0], add=True)
        return carry

    lax.fori_loop(c * (NCH // 2), (c + 1) * (NCH // 2), chunk_body, 0,
                  unroll=False)
    plsc.subcore_barrier()
    pltpu.sync_copy(degacc.at[pl.ds(s * NSEG, NSEG)],
                    degout.at[c].at[pl.ds(s * NSEG, NSEG)])


_deg_call = pl.kernel(
    _deg_sc_body,
    out_type=jax.ShapeDtypeStruct((NSC, EPT), jnp.float32),
    mesh=plsc.VectorSubcoreMesh(core_axis_name="c", subcore_axis_name="s"),
    scratch_types=[
        pltpu.VMEM((NCH, CHUNK), jnp.int32),
        pltpu.VMEM((CHUNK,), jnp.int32),
        pltpu.VMEM((CHUNK,), jnp.float32),
        pltpu.VMEM((NSEG,), jnp.float32),
        pltpu.VMEM_SHARED((EPT,), jnp.float32),
    ],
)


def _edge_layout(edge_index):
    src = edge_index[0].reshape(NTILE, E // NTILE)
    dst = edge_index[1].reshape(NTILE, E // NTILE)
    pad = EPT - E // NTILE
    srcp = jnp.pad(src, ((0, 0), (0, pad)))
    dstp = jnp.pad(dst, ((0, 0), (0, pad)), constant_values=N)
    base = jnp.array([0, N], jnp.int32)[:, None, None]
    srcidx = (srcp[None] + base).reshape(NSC * NTILE, NCH, CHUNK)
    dstidx = dstp.reshape(NTILE, NCH, CHUNK)
    return srcidx, dstidx


def _degree_sc(dstidx):
    return _deg_call(dstidx)[:, :, None]


def kernel(x, edge_index, W0_0, Wz_0, Uz_0, bz_0, Wr_0, Ur_0, br_0, Wh_0, Uh_0, bh_0,
           W0_1, Wz_1, Uz_1, bz_1, Wr_1, Ur_1, br_1, Wh_1, Uh_1, bh_1):
    src = edge_index[0]
    dst = edge_index[1]
    H0, H1 = _gru_evolve(
        (W0_0, Wz_0, Uz_0, bz_0, Wr_0, Ur_0, br_0, Wh_0, Uh_0, bh_0),
        (W0_1, Wz_1, Uz_1, bz_1, Wr_1, Ur_1, br_1, Wh_1, Uh_1, bh_1))
    srcidx, dstidx = _edge_layout(edge_index)
    deg2 = _degree_sc(dstidx)
    hw0p = _stage_a(x.reshape(N, T * D), H0, deg2)
    agg0 = _spmm_sc(hw0p, srcidx, dstidx)
    hw1p = _stage_b(agg0, hw0p, H1, deg2)
    agg1 = _spmm_sc(hw1p, srcidx, dstidx)
    return _stage_c(agg1, hw1p, deg2)


# speedup vs baseline: 1.6232x; 1.4729x over previous
import functools

import jax
import jax.numpy as jnp
from jax import lax
from jax.experimental import pallas as pl
from jax.experimental.pallas import tpu as pltpu
from jax.experimental.pallas import tpu_sc as plsc

N = 10000
E = 160000
T = 8
D = 256
HALF = 128
NBLK = 10
RB = N // NBLK

NSC = 2
NTILE = 16
CHUNK = 128
NCH = 80
EPT = NCH * CHUNK
ACC_ROWS = 10240
RPT = ACC_ROWS // NTILE


def _gru_body(*refs):
    outs = refs[20:]
    for c in range(2):
        (w0, wz, uz, bz, wr, ur, br, wh, uh, bh) = refs[10 * c:10 * (c + 1)]
        H = w0[...]
        Wz, Uz, Wr, Ur = wz[...], uz[...], wr[...], ur[...]
        Wh = wh[...]
        Uh = uh[...]
        bzv = bz[...]
        brv = br[...]
        bhv = bh[...]

        def mm(a, b):
            return jnp.dot(a, b, preferred_element_type=jnp.float32)

        for _ in range(T):
            z = jax.nn.sigmoid(mm(Wz, H) + mm(Uz, H) + bzv)
            r = jax.nn.sigmoid(mm(Wr, H) + mm(Ur, H) + brv)
            ht = jnp.tanh(mm(Wh, H) + mm(Uh, r * H) + bhv)
            H = (1.0 - z) * H + z * ht
        outs[c][...] = H


def _gru_evolve(params0, params1):
    out = pl.pallas_call(
        _gru_body,
        out_shape=[jax.ShapeDtypeStruct((D, D), jnp.float32)] * 2,
    )(*params0, *params1)
    return out


def _stage_a_body(x_ref, h_ref, deg_ref, out_ref):
    xb = x_ref[...]
    d = deg_ref[0, :, :] + deg_ref[1, :, :] + 1.0
    dinv = lax.rsqrt(jnp.maximum(d, 1.0))
    out_ref[0] = jnp.dot(xb, h_ref[...], preferred_element_type=jnp.float32) * dinv


def _stage_a(x2d, H0, deg2):
    return pl.pallas_call(
        _stage_a_body,
        grid=(NBLK, 2),
        in_specs=[
            pl.BlockSpec((RB, D), lambda i, h: (i, T - 1)),
            pl.BlockSpec((D, HALF), lambda i, h: (0, h)),
            pl.BlockSpec((2, RB, 1), lambda i, h: (0, i, 0)),
        ],
        out_specs=pl.BlockSpec((1, RB, HALF), lambda i, h: (h, i, 0)),
        out_shape=jax.ShapeDtypeStruct((2, N, HALF), jnp.float32),
    )(x2d, H0, deg2)


def _stage_b_body(agg_ref, hw_ref, h_ref, deg_ref, out_ref):
    d = deg_ref[0, :, :] + deg_ref[1, :, :] + 1.0
    dinv = lax.rsqrt(jnp.maximum(d, 1.0))
    o0 = jnp.maximum(dinv * (agg_ref[0] + hw_ref[0]), 0.0)
    o1 = jnp.maximum(dinv * (agg_ref[1] + hw_ref[1]), 0.0)
    acc = (jnp.dot(o0, h_ref[:HALF, :], preferred_element_type=jnp.float32)
           + jnp.dot(o1, h_ref[HALF:, :], preferred_element_type=jnp.float32))
    out_ref[0] = acc * dinv


def _stage_b(agg0, hw0p, H1, deg2):
    return pl.pallas_call(
        _stage_b_body,
        grid=(NBLK, 2),
        in_specs=[
            pl.BlockSpec((2, RB, HALF), lambda i, h: (0, i, 0)),
            pl.BlockSpec((2, RB, HALF), lambda i, h: (0, i, 0)),
            pl.BlockSpec((D, HALF), lambda i, h: (0, h)),
            pl.BlockSpec((2, RB, 1), lambda i, h: (0, i, 0)),
        ],
        out_specs=pl.BlockSpec((1, RB, HALF), lambda i, h: (h, i, 0)),
        out_shape=jax.ShapeDtypeStruct((2, N, HALF), jnp.float32),
    )(agg0, hw0p, H1, deg2)


def _stage_c_body(agg_ref, hw_ref, deg_ref, out_ref):
    d = deg_ref[0, :, :] + deg_ref[1, :, :] + 1.0
    dinv = lax.rsqrt(jnp.maximum(d, 1.0))
    out_ref[...] = jnp.maximum(dinv * (agg_ref[0] + hw_ref[0]), 0.0)


def _stage_c(agg1, hw1p, deg2):
    return pl.pallas_call(
        _stage_c_body,
        grid=(NBLK, 2),
        in_specs=[
            pl.BlockSpec((1, RB, HALF), lambda i, h: (h, i, 0)),
            pl.BlockSpec((1, RB, HALF), lambda i, h: (h, i, 0)),
            pl.BlockSpec((2, RB, 1), lambda i, h: (0, i, 0)),
        ],
        out_specs=pl.BlockSpec((RB, HALF), lambda i, h: (i, h)),
        out_shape=jax.ShapeDtypeStruct((N, D), jnp.float32),
    )(agg1, hw1p, deg2)


EDGES_PT = E // NTILE
NFULL = EDGES_PT // CHUNK
TAIL = EDGES_PT - NFULL * CHUNK


def _zero_rows(buf, nrows):
    zero16 = jnp.zeros((16,), jnp.float32)
    ncol = buf.shape[1] // 16

    def zrow(r, carry):
        for k in range(ncol):
            buf[r, pl.ds(k * 16, 16)] = zero16
        return carry

    lax.fori_loop(0, nrows, zrow, 0, unroll=False)


def _spmm_sc_body(table, ei, out,
                  s0, d0, s1, d1, st, dt, rows0, rows1, acc,
                  semi0, semi1, semg0, semg1):
    c = lax.axis_index("c")
    s = lax.axis_index("s")
    soff = s * EDGES_PT
    doff = E + s * EDGES_PT

    def idx_start(j, sb, db, sem):
        pltpu.async_copy(ei.at[pl.ds(soff + j * CHUNK, CHUNK)], sb, sem)
        pltpu.async_copy(ei.at[pl.ds(doff + j * CHUNK, CHUNK)], db, sem)

    def idx_wait(sb, db, sem):
        pltpu.make_async_copy(ei.at[pl.ds(0, CHUNK)], sb, sem).wait()
        pltpu.make_async_copy(ei.at[pl.ds(0, CHUNK)], db, sem).wait()

    def gather_start(sb, rbuf, sem):
        pltpu.async_copy(table.at[c].at[sb], rbuf, sem)

    def gather_wait(rbuf, sem):
        pltpu.make_async_copy(table.at[c].at[s0], rbuf, sem).wait()

    idx_start(0, s0, d0, semi0)
    idx_start(1, s1, d1, semi1)
    _zero_rows(rows0, CHUNK)
    for q in range(5):
        pltpu.sync_copy(rows0, acc.at[pl.ds(s * RPT + q * CHUNK, CHUNK)])
    plsc.subcore_barrier()

    idx_wait(s0, d0, semi0)
    gather_start(s0, rows0, semg0)
    idx_wait(s1, d1, semi1)
    gather_start(s1, rows1, semg1)

    def body(g, carry):
        j = 2 * g
        gather_wait(rows0, semg0)
        pltpu.sync_copy(rows0, acc.at[d0], add=True)
        idx_start(j + 2, s0, d0, semi0)
        gather_wait(rows1, semg1)
        pltpu.sync_copy(rows1, acc.at[d1], add=True)
        idx_start(j + 3, s1, d1, semi1)
        idx_wait(s0, d0, semi0)
        gather_start(s0, rows0, semg0)
        idx_wait(s1, d1, semi1)
        gather_start(s1, rows1, semg1)
        return carry

    lax.fori_loop(0, (NFULL - 2) // 2, body, 0, unroll=False)
    gather_wait(rows0, semg0)
    pltpu.sync_copy(rows0, acc.at[d0], add=True)
    gather_wait(rows1, semg1)
    pltpu.sync_copy(rows1, acc.at[d1], add=True)

    pltpu.sync_copy(ei.at[pl.ds(soff + NFULL * CHUNK, TAIL)], st)
    pltpu.sync_copy(ei.at[pl.ds(doff + NFULL * CHUNK, TAIL)], dt)
    pltpu.async_copy(table.at[c].at[st], rows0.at[pl.ds(0, TAIL)], semg0)
    pltpu.make_async_copy(table.at[c].at[st], rows0.at[pl.ds(0, TAIL)],
                          semg0).wait()
    pltpu.sync_copy(rows0.at[pl.ds(0, TAIL)], acc.at[dt], add=True)

    plsc.subcore_barrier()
    pltpu.sync_copy(acc.at[pl.ds(s * RPT, RPT)],
                    out.at[c].at[pl.ds(s * RPT, RPT)])


_spmm_call = pl.kernel(
    _spmm_sc_body,
    out_type=jax.ShapeDtypeStruct((NSC, ACC_ROWS, HALF), jnp.float32),
    mesh=plsc.VectorSubcoreMesh(core_axis_name="c", subcore_axis_name="s"),
    scratch_types=[
        pltpu.VMEM((CHUNK,), jnp.int32),
        pltpu.VMEM((CHUNK,), jnp.int32),
        pltpu.VMEM((CHUNK,), jnp.int32),
        pltpu.VMEM((CHUNK,), jnp.int32),
        pltpu.VMEM((TAIL,), jnp.int32),
        pltpu.VMEM((TAIL,), jnp.int32),
        pltpu.VMEM((CHUNK, HALF), jnp.float32),
        pltpu.VMEM((CHUNK, HALF), jnp.float32),
        pltpu.VMEM_SHARED((ACC_ROWS, HALF), jnp.float32),
        pltpu.SemaphoreType.DMA,
        pltpu.SemaphoreType.DMA,
        pltpu.SemaphoreType.DMA,
        pltpu.SemaphoreType.DMA,
    ],
)


def _spmm_sc(table2, ei):
    return _spmm_call(table2, ei)


EPT = 10240
NSEG = EPT // NTILE


def _deg_sc_body(ei, degout, d0, dt, ones, zbuf, degacc):
    c = lax.axis_index("c")
    s = lax.axis_index("s")
    doff = E + s * EDGES_PT
    one16 = jnp.ones((16,), jnp.float32)
    zero16 = jnp.zeros((16,), jnp.float32)
    for k in range(CHUNK // 16):
        ones[pl.ds(16 * k, 16)] = one16

    def zrow(r, carry):
        zbuf[pl.ds(r * 16, 16)] = zero16
        return carry

    lax.fori_loop(0, NSEG // 16, zrow, 0, unroll=False)
    pltpu.sync_copy(zbuf, degacc.at[pl.ds(s * NSEG, NSEG)])
    plsc.subcore_barrier()

    def chunk_body(j, carry):
        pltpu.sync_copy(ei.at[pl.ds(doff + j * CHUNK, CHUNK)], d0)
        pltpu.sync_copy(ones, degacc.at[d0], add=True)
        return carry

    lax.fori_loop(c * (NFULL // 2), (c + 1) * (NFULL // 2), chunk_body, 0,
                  unroll=False)

    @pl.when(c == 1)
    def _():
        pltpu.sync_copy(ei.at[pl.ds(doff + NFULL * CHUNK, TAIL)], dt)
        pltpu.sync_copy(ones.at[pl.ds(0, TAIL)], degacc.at[dt], add=True)

    plsc.subcore_barrier()
    pltpu.sync_copy(degacc.at[pl.ds(s * NSEG, NSEG)],
                    degout.at[c].at[pl.ds(s * NSEG, NSEG)])


_deg_call = pl.kernel(
    _deg_sc_body,
    out_type=jax.ShapeDtypeStruct((NSC, EPT), jnp.float32),
    mesh=plsc.VectorSubcoreMesh(core_axis_name="c", subcore_axis_name="s"),
    scratch_types=[
        pltpu.VMEM((CHUNK,), jnp.int32),
        pltpu.VMEM((TAIL,), jnp.int32),
        pltpu.VMEM((CHUNK,), jnp.float32),
        pltpu.VMEM((NSEG,), jnp.float32),
        pltpu.VMEM_SHARED((EPT,), jnp.float32),
    ],
)


def _degree_sc(ei):
    return _deg_call(ei)[:, :, None]


def kernel(x, edge_index, W0_0, Wz_0, Uz_0, bz_0, Wr_0, Ur_0, br_0, Wh_0, Uh_0, bh_0,
           W0_1, Wz_1, Uz_1, bz_1, Wr_1, Ur_1, br_1, Wh_1, Uh_1, bh_1):
    src = edge_index[0]
    dst = edge_index[1]
    H0, H1 = _gru_evolve(
        (W0_0, Wz_0, Uz_0, bz_0, Wr_0, Ur_0, br_0, Wh_0, Uh_0, bh_0),
        (W0_1, Wz_1, Uz_1, bz_1, Wr_1, Ur_1, br_1, Wh_1, Uh_1, bh_1))
    ei = edge_index.reshape(2 * E)
    deg2 = _degree_sc(ei)
    hw0p = _stage_a(x.reshape(N, T * D), H0, deg2)
    agg0 = _spmm_sc(hw0p, ei)
    hw1p = _stage_b(agg0, hw0p, H1, deg2)
    agg1 = _spmm_sc(hw1p, ei)
    return _stage_c(agg1, hw1p, deg2)


# speedup vs baseline: 1.6262x; 1.0019x over previous
import functools

import jax
import jax.numpy as jnp
from jax import lax
from jax.experimental import pallas as pl
from jax.experimental.pallas import tpu as pltpu
from jax.experimental.pallas import tpu_sc as plsc

N = 10000
E = 160000
T = 8
D = 256
HALF = 128
NBLK = 10
RB = N // NBLK

NSC = 2
NTILE = 16
CHUNK = 128
NCH = 80
EPT = NCH * CHUNK
ACC_ROWS = 10240
RPT = ACC_ROWS // NTILE


def _gru_body(*refs):
    outs = refs[20:]
    for c in range(2):
        (w0, wz, uz, bz, wr, ur, br, wh, uh, bh) = refs[10 * c:10 * (c + 1)]
        H = w0[...]
        Wz, Uz, Wr, Ur = wz[...], uz[...], wr[...], ur[...]
        Wh = wh[...]
        Uh = uh[...]
        bzv = bz[...]
        brv = br[...]
        bhv = bh[...]

        def mm(a, b):
            return jnp.dot(a, b, preferred_element_type=jnp.float32)

        for _ in range(T):
            z = jax.nn.sigmoid(mm(Wz, H) + mm(Uz, H) + bzv)
            r = jax.nn.sigmoid(mm(Wr, H) + mm(Ur, H) + brv)
            ht = jnp.tanh(mm(Wh, H) + mm(Uh, r * H) + bhv)
            H = (1.0 - z) * H + z * ht
        outs[c][...] = H


def _gru_evolve(params0, params1):
    out = pl.pallas_call(
        _gru_body,
        out_shape=[jax.ShapeDtypeStruct((D, D), jnp.float32)] * 2,
    )(*params0, *params1)
    return out


def _stage_a_body(x_ref, h_ref, deg_ref, out_ref):
    xb = x_ref[...]
    d = deg_ref[0, :, :] + deg_ref[1, :, :] + 1.0
    dinv = lax.rsqrt(jnp.maximum(d, 1.0))
    out_ref[0] = jnp.dot(xb, h_ref[...], preferred_element_type=jnp.float32) * dinv


def _stage_a(x2d, H0, deg2):
    return pl.pallas_call(
        _stage_a_body,
        grid=(NBLK, 2),
        in_specs=[
            pl.BlockSpec((RB, D), lambda i, h: (i, T - 1)),
            pl.BlockSpec((D, HALF), lambda i, h: (0, h)),
            pl.BlockSpec((2, RB, 1), lambda i, h: (0, i, 0)),
        ],
        out_specs=pl.BlockSpec((1, RB, HALF), lambda i, h: (h, i, 0)),
        out_shape=jax.ShapeDtypeStruct((2, N, HALF), jnp.float32),
    )(x2d, H0, deg2)


def _stage_b_body(agg_ref, hw_ref, h_ref, deg_ref, out_ref):
    d = deg_ref[0, :, :] + deg_ref[1, :, :] + 1.0
    dinv = lax.rsqrt(jnp.maximum(d, 1.0))
    o0 = jnp.maximum(dinv * (agg_ref[0] + hw_ref[0]), 0.0)
    o1 = jnp.maximum(dinv * (agg_ref[1] + hw_ref[1]), 0.0)
    acc = (jnp.dot(o0, h_ref[:HALF, :], preferred_element_type=jnp.float32)
           + jnp.dot(o1, h_ref[HALF:, :], preferred_element_type=jnp.float32))
    out_ref[0] = acc * dinv


def _stage_b(agg0, hw0p, H1, deg2):
    return pl.pallas_call(
        _stage_b_body,
        grid=(NBLK, 2),
        in_specs=[
            pl.BlockSpec((2, RB, HALF), lambda i, h: (0, i, 0)),
            pl.BlockSpec((2, RB, HALF), lambda i, h: (0, i, 0)),
            pl.BlockSpec((D, HALF), lambda i, h: (0, h)),
            pl.BlockSpec((2, RB, 1), lambda i, h: (0, i, 0)),
        ],
        out_specs=pl.BlockSpec((1, RB, HALF), lambda i, h: (h, i, 0)),
        out_shape=jax.ShapeDtypeStruct((2, N, HALF), jnp.float32),
    )(agg0, hw0p, H1, deg2)


def _stage_c_body(agg_ref, hw_ref, deg_ref, out_ref):
    d = deg_ref[0, :, :] + deg_ref[1, :, :] + 1.0
    dinv = lax.rsqrt(jnp.maximum(d, 1.0))
    out_ref[...] = jnp.maximum(dinv * (agg_ref[0] + hw_ref[0]), 0.0)


def _stage_c(agg1, hw1p, deg2):
    return pl.pallas_call(
        _stage_c_body,
        grid=(NBLK, 2),
        in_specs=[
            pl.BlockSpec((1, RB, HALF), lambda i, h: (h, i, 0)),
            pl.BlockSpec((1, RB, HALF), lambda i, h: (h, i, 0)),
            pl.BlockSpec((2, RB, 1), lambda i, h: (0, i, 0)),
        ],
        out_specs=pl.BlockSpec((RB, HALF), lambda i, h: (i, h)),
        out_shape=jax.ShapeDtypeStruct((N, D), jnp.float32),
    )(agg1, hw1p, deg2)


EDGES_PT = E // NTILE
NFULL = EDGES_PT // CHUNK
TAIL = EDGES_PT - NFULL * CHUNK


def _zero_rows(buf, nrows):
    zero16 = jnp.zeros((16,), jnp.float32)
    ncol = buf.shape[1] // 16

    def zrow(r, carry):
        for k in range(ncol):
            buf[r, pl.ds(k * 16, 16)] = zero16
        return carry

    lax.fori_loop(0, nrows, zrow, 0, unroll=False)


def _spmm_sc_body(table, ei, out,
                  s0, d0, s1, d1, st, dt, rows0, rows1, acc,
                  semi0, semi1, semg0, semg1):
    c = lax.axis_index("c")
    s = lax.axis_index("s")
    soff = s * EDGES_PT
    doff = E + s * EDGES_PT

    def idx_start(j, sb, db, sem):
        pltpu.async_copy(ei.at[pl.ds(soff + j * CHUNK, CHUNK)], sb, sem)
        pltpu.async_copy(ei.at[pl.ds(doff + j * CHUNK, CHUNK)], db, sem)

    def idx_wait(sb, db, sem):
        pltpu.make_async_copy(ei.at[pl.ds(0, CHUNK)], sb, sem).wait()
        pltpu.make_async_copy(ei.at[pl.ds(0, CHUNK)], db, sem).wait()

    def gather_start(sb, rbuf, sem):
        pltpu.async_copy(table.at[c].at[sb], rbuf, sem)

    def gather_wait(rbuf, sem):
        pltpu.make_async_copy(table.at[c].at[s0], rbuf, sem).wait()

    idx_start(0, s0, d0, semi0)
    idx_start(1, s1, d1, semi1)
    _zero_rows(rows0, CHUNK)
    for q in range(5):
        pltpu.sync_copy(rows0, acc.at[pl.ds(s * RPT + q * CHUNK, CHUNK)])
    plsc.subcore_barrier()

    idx_wait(s0, d0, semi0)
    gather_start(s0, rows0, semg0)
    idx_wait(s1, d1, semi1)
    gather_start(s1, rows1, semg1)

    def body(g, carry):
        j = 2 * g
        gather_wait(rows0, semg0)
        pltpu.sync_copy(rows0, acc.at[d0], add=True)
        idx_start(j + 2, s0, d0, semi0)
        gather_wait(rows1, semg1)
        pltpu.sync_copy(rows1, acc.at[d1], add=True)
        idx_start(j + 3, s1, d1, semi1)
        idx_wait(s0, d0, semi0)
        gather_start(s0, rows0, semg0)
        idx_wait(s1, d1, semi1)
        gather_start(s1, rows1, semg1)
        return carry

    lax.fori_loop(0, (NFULL - 2) // 2, body, 0, unroll=False)
    gather_wait(rows0, semg0)
    pltpu.sync_copy(rows0, acc.at[d0], add=True)
    gather_wait(rows1, semg1)
    pltpu.sync_copy(rows1, acc.at[d1], add=True)

    pltpu.sync_copy(ei.at[pl.ds(soff + NFULL * CHUNK, TAIL)], st)
    pltpu.sync_copy(ei.at[pl.ds(doff + NFULL * CHUNK, TAIL)], dt)
    pltpu.async_copy(table.at[c].at[st], rows0.at[pl.ds(0, TAIL)], semg0)
    pltpu.make_async_copy(table.at[c].at[st], rows0.at[pl.ds(0, TAIL)],
                          semg0).wait()
    pltpu.sync_copy(rows0.at[pl.ds(0, TAIL)], acc.at[dt], add=True)

    plsc.subcore_barrier()
    pltpu.sync_copy(acc.at[pl.ds(s * RPT, RPT)],
                    out.at[c].at[pl.ds(s * RPT, RPT)])


_spmm_call = pl.kernel(
    _spmm_sc_body,
    out_type=jax.ShapeDtypeStruct((NSC, ACC_ROWS, HALF), jnp.float32),
    mesh=plsc.VectorSubcoreMesh(core_axis_name="c", subcore_axis_name="s"),
    scratch_types=[
        pltpu.VMEM((CHUNK,), jnp.int32),
        pltpu.VMEM((CHUNK,), jnp.int32),
        pltpu.VMEM((CHUNK,), jnp.int32),
        pltpu.VMEM((CHUNK,), jnp.int32),
        pltpu.VMEM((TAIL,), jnp.int32),
        pltpu.VMEM((TAIL,), jnp.int32),
        pltpu.VMEM((CHUNK, HALF), jnp.float32),
        pltpu.VMEM((CHUNK, HALF), jnp.float32),
        pltpu.VMEM_SHARED((ACC_ROWS, HALF), jnp.float32),
        pltpu.SemaphoreType.DMA,
        pltpu.SemaphoreType.DMA,
        pltpu.SemaphoreType.DMA,
        pltpu.SemaphoreType.DMA,
    ],
)


def _spmm_sc(table2, ei):
    return _spmm_call(table2, ei)


EPT = 10240
NSEG = EPT // NTILE


def _deg_sc_body(ei, degout, d0, dt, ones, zbuf, degacc):
    c = lax.axis_index("c")
    s = lax.axis_index("s")
    doff = E + s * EDGES_PT
    one16 = jnp.ones((16,), jnp.float32)
    zero16 = jnp.zeros((16,), jnp.float32)
    for k in range(CHUNK // 16):
        ones[pl.ds(16 * k, 16)] = one16

    def zrow(r, carry):
        zbuf[pl.ds(r * 16, 16)] = zero16
        return carry

    lax.fori_loop(0, NSEG // 16, zrow, 0, unroll=False)
    pltpu.sync_copy(zbuf, degacc.at[pl.ds(s * NSEG, NSEG)])
    plsc.subcore_barrier()

    def chunk_body(j, carry):
        pltpu.sync_copy(ei.at[pl.ds(doff + j * CHUNK, CHUNK)], d0)
        pltpu.sync_copy(ones, degacc.at[d0], add=True)
        return carry

    lax.fori_loop(c * (NFULL // 2), (c + 1) * (NFULL // 2), chunk_body, 0,
                  unroll=False)

    @pl.when(c == 1)
    def _():
        pltpu.sync_copy(ei.at[pl.ds(doff + NFULL * CHUNK, TAIL)], dt)
        pltpu.sync_copy(ones.at[pl.ds(0, TAIL)], degacc.at[dt], add=True)

    plsc.subcore_barrier()
    pltpu.sync_copy(degacc.at[pl.ds(s * NSEG, NSEG)],
                    degout.at[c].at[pl.ds(s * NSEG, NSEG)])


_deg_call = pl.kernel(
    _deg_sc_body,
    out_type=jax.ShapeDtypeStruct((NSC, EPT), jnp.float32),
    mesh=plsc.VectorSubcoreMesh(core_axis_name="c", subcore_axis_name="s"),
    scratch_types=[
        pltpu.VMEM((CHUNK,), jnp.int32),
        pltpu.VMEM((TAIL,), jnp.int32),
        pltpu.VMEM((CHUNK,), jnp.float32),
        pltpu.VMEM((NSEG,), jnp.float32),
        pltpu.VMEM_SHARED((EPT,), jnp.float32),
    ],
)


def _degree_sc(ei):
    return _deg_call(ei)[:, :N, None]


def kernel(x, edge_index, W0_0, Wz_0, Uz_0, bz_0, Wr_0, Ur_0, br_0, Wh_0, Uh_0, bh_0,
           W0_1, Wz_1, Uz_1, bz_1, Wr_1, Ur_1, br_1, Wh_1, Uh_1, bh_1):
    src = edge_index[0]
    dst = edge_index[1]
    H0, H1 = _gru_evolve(
        (W0_0, Wz_0, Uz_0, bz_0, Wr_0, Ur_0, br_0, Wh_0, Uh_0, bh_0),
        (W0_1, Wz_1, Uz_1, bz_1, Wr_1, Ur_1, br_1, Wh_1, Uh_1, bh_1))
    ei = edge_index.reshape(2 * E)
    deg2 = _degree_sc(ei)
    hw0p = _stage_a(x.reshape(N, T * D), H0, deg2)
    agg0 = _spmm_sc(hw0p, ei)
    hw1p = _stage_b(agg0, hw0p, H1, deg2)
    agg1 = _spmm_sc(hw1p, ei)
    return _stage_c(agg1, hw1p, deg2)
